# Initial kernel scaffold; baseline (speedup 1.0000x reference)
import functools

import jax
import jax.numpy as jnp
from jax import lax
from jax.experimental import pallas as pl
from jax.experimental.pallas import tpu as pltpu

HID = 256
HEADS = 4
DH = HID // HEADS
N_TERM = 4096
BLK = 256
N_BLKS = N_TERM // BLK
CNTW = 16


def _dense_a_body(sums_ref, cnts_ref, x_ref, wl_ref, wr_ref, bsum_ref,
                  lng_ref, lnb_ref, inw_ref, inb_ref,
                  h_ref, q_ref, k_ref, v_ref):
    x = x_ref[...]
    acc = x + bsum_ref[...]
    acc = acc + lax.dot_general(x, wr_ref[...], (((1,), (1,)), ((), ())),
                                preferred_element_type=jnp.float32)
    for r in range(4):
        c = cnts_ref[r, :, 0][:, None]
        inv = 1.0 / jnp.maximum(c, 1.0)
        mean = sums_ref[r] * inv
        acc = acc + lax.dot_general(mean, wl_ref[r], (((1,), (1,)), ((), ())),
                                    preferred_element_type=jnp.float32)
    mu = jnp.mean(acc, axis=1, keepdims=True)
    var = jnp.mean(acc * acc, axis=1, keepdims=True) - mu * mu
    h = (acc - mu) * lax.rsqrt(var + 1e-5) * lng_ref[...] + lnb_ref[...]
    h_ref[...] = h
    qkv = lax.dot_general(h, inw_ref[...], (((1,), (1,)), ((), ())),
                          preferred_element_type=jnp.float32) + inb_ref[...]
    q_ref[...] = qkv[:, 0:HID]
    k_ref[...] = qkv[:, HID:2 * HID]
    v_ref[...] = qkv[:, 2 * HID:3 * HID]


def _dense_a(sums, cnts, x_term, Wl4, Wr_sum, b_sum, ln_g, ln_b, in_w, in_b):
    spec_row = pl.BlockSpec((BLK, HID), lambda i: (i, 0))
    return pl.pallas_call(
        _dense_a_body,
        grid=(N_BLKS,),
        in_specs=[
            pl.BlockSpec((4, BLK, HID), lambda i: (0, i, 0)),
            pl.BlockSpec((4, BLK, CNTW), lambda i: (0, i, 0)),
            spec_row,
            pl.BlockSpec((4, HID, HID), lambda i: (0, 0, 0)),
            pl.BlockSpec((HID, HID), lambda i: (0, 0)),
            pl.BlockSpec((1, HID), lambda i: (0, 0)),
            pl.BlockSpec((1, HID), lambda i: (0, 0)),
            pl.BlockSpec((3 * HID, HID), lambda i: (0, 0)),
            pl.BlockSpec((1, 3 * HID), lambda i: (0, 0)),
        ],
        out_specs=[spec_row, spec_row, spec_row, spec_row],
        out_shape=[jax.ShapeDtypeStruct((N_TERM, HID), jnp.float32)] * 4,
    )(sums, cnts, x_term, Wl4, Wr_sum, b_sum, ln_g, ln_b, in_w, in_b)


def _attn_body(q_ref, k_ref, v_ref, h_ref, ow_ref, ob_ref, pw_ref, pb_ref,
               out_ref):
    q = q_ref[...]
    scale = 1.0 / (DH ** 0.5)
    outs = []
    for hh in range(HEADS):
        qh = q[:, hh * DH:(hh + 1) * DH] * scale
        kh = k_ref[:, hh * DH:(hh + 1) * DH]
        vh = v_ref[:, hh * DH:(hh + 1) * DH]
        s = lax.dot_general(qh, kh, (((1,), (1,)), ((), ())),
                            preferred_element_type=jnp.float32)
        m = jnp.max(s, axis=1, keepdims=True)
        p = jnp.exp(s - m)
        denom = jnp.sum(p, axis=1, keepdims=True)
        attn = p / denom
        outs.append(lax.dot_general(attn, vh, (((1,), (0,)), ((), ())),
                                    preferred_element_type=jnp.float32))
    o = jnp.concatenate(outs, axis=1)
    a = lax.dot_general(o, ow_ref[...], (((1,), (1,)), ((), ())),
                        preferred_element_type=jnp.float32) + ob_ref[...]
    f = lax.dot_general(a, pw_ref[...], (((1,), (1,)), ((), ())),
                        preferred_element_type=jnp.float32) + pb_ref[...]
    out_ref[...] = jnp.maximum(f, 0.0) + h_ref[...]


def _attn(q, k, v, h, out_w, out_b, post_w, post_b):
    spec_row = pl.BlockSpec((BLK, HID), lambda i: (i, 0))
    spec_full = pl.BlockSpec((N_TERM, HID), lambda i: (0, 0))
    return pl.pallas_call(
        _attn_body,
        grid=(N_BLKS,),
        in_specs=[
            spec_row, spec_full, spec_full, spec_row,
            pl.BlockSpec((HID, HID), lambda i: (0, 0)),
            pl.BlockSpec((1, HID), lambda i: (0, 0)),
            pl.BlockSpec((HID, HID), lambda i: (0, 0)),
            pl.BlockSpec((1, HID), lambda i: (0, 0)),
        ],
        out_specs=spec_row,
        out_shape=jax.ShapeDtypeStruct((N_TERM, HID), jnp.float32),
    )(q, k, v, h, out_w, out_b, post_w, post_b)


def _segment_sums(x_term, x_symbol, x_var, has_arg_src, has_arg_dst,
                  symbol_of_src, symbol_of_dst, var_occ_src, var_occ_dst):
    rels = [
        (x_term, has_arg_dst, has_arg_src),
        (x_term, has_arg_src, has_arg_dst),
        (x_symbol, symbol_of_dst, symbol_of_src),
        (x_var, var_occ_src, var_occ_dst),
    ]
    sums, cnts = [], []
    for tab, src, dst in rels:
        msg = jnp.take(tab, src, axis=0)
        sums.append(jax.ops.segment_sum(msg, dst, num_segments=N_TERM))
        c = jax.ops.segment_sum(jnp.ones((src.shape[0],), jnp.float32), dst,
                                num_segments=N_TERM)
        cnts.append(jnp.broadcast_to(c[:, None], (N_TERM, CNTW)))
    return jnp.stack(sums), jnp.stack(cnts)


def kernel(x_term, x_symbol, x_var, has_arg_src, has_arg_dst, symbol_of_src,
           symbol_of_dst, var_occ_src, var_occ_dst, Wl, bl, Wr, ln_g, ln_b,
           attn_in_w, attn_in_b, attn_out_w, attn_out_b, post_w, post_b):
    sums, cnts = _segment_sums(x_term, x_symbol, x_var, has_arg_src,
                               has_arg_dst, symbol_of_src, symbol_of_dst,
                               var_occ_src, var_occ_dst)
    Wr_sum = Wr[0] + Wr[1] + Wr[2] + Wr[3]
    b_sum = (bl[0] + bl[1] + bl[2] + bl[3]).reshape(1, HID)
    h, q, k, v = _dense_a(sums, cnts, x_term, Wl[:4], Wr_sum, b_sum,
                          ln_g.reshape(1, HID), ln_b.reshape(1, HID),
                          attn_in_w, attn_in_b.reshape(1, 3 * HID))
    return _attn(q, k, v, h, attn_out_w, attn_out_b.reshape(1, HID),
                 post_w, post_b.reshape(1, HID))


# speedup vs baseline: 1.0654x; 1.0654x over previous
import functools

import jax
import jax.numpy as jnp
from jax import lax
from jax.experimental import pallas as pl
from jax.experimental.pallas import tpu as pltpu

HID = 256
HEADS = 4
DH = HID // HEADS
N_TERM = 4096
BLK = 256
N_BLKS = N_TERM // BLK
CNTW = 16


def _dense_a_body(sums_ref, cnts_ref, x_ref, wl_ref, wr_ref, bsum_ref,
                  lng_ref, lnb_ref, inw_ref, inb_ref,
                  h_ref, q_ref, k_ref, v_ref):
    x = x_ref[...]
    acc = x + bsum_ref[...]
    acc = acc + lax.dot_general(x, wr_ref[...], (((1,), (1,)), ((), ())),
                                preferred_element_type=jnp.float32)
    for r in range(4):
        c = cnts_ref[r, :, 0][:, None]
        inv = 1.0 / jnp.maximum(c, 1.0)
        mean = sums_ref[r] * inv
        acc = acc + lax.dot_general(mean, wl_ref[r], (((1,), (1,)), ((), ())),
                                    preferred_element_type=jnp.float32)
    mu = jnp.mean(acc, axis=1, keepdims=True)
    var = jnp.mean(acc * acc, axis=1, keepdims=True) - mu * mu
    h = (acc - mu) * lax.rsqrt(var + 1e-5) * lng_ref[...] + lnb_ref[...]
    h_ref[...] = h
    qkv = lax.dot_general(h, inw_ref[...], (((1,), (1,)), ((), ())),
                          preferred_element_type=jnp.float32) + inb_ref[...]
    q_ref[...] = qkv[:, 0:HID]
    k_ref[...] = qkv[:, HID:2 * HID]
    v_ref[...] = qkv[:, 2 * HID:3 * HID]


def _dense_a(sums, cnts, x_term, Wl4, Wr_sum, b_sum, ln_g, ln_b, in_w, in_b):
    spec_row = pl.BlockSpec((BLK, HID), lambda i: (i, 0))
    return pl.pallas_call(
        _dense_a_body,
        grid=(N_BLKS,),
        in_specs=[
            pl.BlockSpec((4, BLK, HID), lambda i: (0, i, 0)),
            pl.BlockSpec((4, BLK, CNTW), lambda i: (0, i, 0)),
            spec_row,
            pl.BlockSpec((4, HID, HID), lambda i: (0, 0, 0)),
            pl.BlockSpec((HID, HID), lambda i: (0, 0)),
            pl.BlockSpec((1, HID), lambda i: (0, 0)),
            pl.BlockSpec((1, HID), lambda i: (0, 0)),
            pl.BlockSpec((1, HID), lambda i: (0, 0)),
            pl.BlockSpec((3 * HID, HID), lambda i: (0, 0)),
            pl.BlockSpec((1, 3 * HID), lambda i: (0, 0)),
        ],
        out_specs=[spec_row, spec_row, spec_row, spec_row],
        out_shape=[jax.ShapeDtypeStruct((N_TERM, HID), jnp.float32)] * 4,
    )(sums, cnts, x_term, Wl4, Wr_sum, b_sum, ln_g, ln_b, in_w, in_b)


def _attn_body(q_ref, k_ref, v_ref, h_ref, ow_ref, ob_ref, pw_ref, pb_ref,
               out_ref):
    q = q_ref[...]
    scale = 1.0 / (DH ** 0.5)
    outs = []
    for hh in range(HEADS):
        qh = q[:, hh * DH:(hh + 1) * DH] * scale
        kh = k_ref[:, hh * DH:(hh + 1) * DH]
        vh = v_ref[:, hh * DH:(hh + 1) * DH]
        s = lax.dot_general(qh, kh, (((1,), (1,)), ((), ())),
                            preferred_element_type=jnp.float32)
        m = jnp.max(s, axis=1, keepdims=True)
        p = jnp.exp(s - m)
        denom = jnp.sum(p, axis=1, keepdims=True)
        attn = p / denom
        outs.append(lax.dot_general(attn, vh, (((1,), (0,)), ((), ())),
                                    preferred_element_type=jnp.float32))
    o = jnp.concatenate(outs, axis=1)
    a = lax.dot_general(o, ow_ref[...], (((1,), (1,)), ((), ())),
                        preferred_element_type=jnp.float32) + ob_ref[...]
    f = lax.dot_general(a, pw_ref[...], (((1,), (1,)), ((), ())),
                        preferred_element_type=jnp.float32) + pb_ref[...]
    out_ref[...] = jnp.maximum(f, 0.0) + h_ref[...]


def _attn(q, k, v, h, out_w, out_b, post_w, post_b):
    spec_row = pl.BlockSpec((BLK, HID), lambda i: (i, 0))
    spec_full = pl.BlockSpec((N_TERM, HID), lambda i: (0, 0))
    return pl.pallas_call(
        _attn_body,
        grid=(N_BLKS,),
        in_specs=[
            spec_row, spec_full, spec_full, spec_row,
            pl.BlockSpec((HID, HID), lambda i: (0, 0)),
            pl.BlockSpec((1, HID), lambda i: (0, 0)),
            pl.BlockSpec((HID, HID), lambda i: (0, 0)),
            pl.BlockSpec((1, HID), lambda i: (0, 0)),
        ],
        out_specs=spec_row,
        out_shape=jax.ShapeDtypeStruct((N_TERM, HID), jnp.float32),
    )(q, k, v, h, out_w, out_b, post_w, post_b)


def _segment_sums(x_term, x_symbol, x_var, has_arg_src, has_arg_dst,
                  symbol_of_src, symbol_of_dst, var_occ_src, var_occ_dst):
    rels = [
        (x_term, has_arg_dst, has_arg_src),
        (x_term, has_arg_src, has_arg_dst),
        (x_symbol, symbol_of_dst, symbol_of_src),
        (x_var, var_occ_src, var_occ_dst),
    ]
    sums, cnts = [], []
    for tab, src, dst in rels:
        msg = jnp.take(tab, src, axis=0)
        sums.append(jax.ops.segment_sum(msg, dst, num_segments=N_TERM))
        c = jax.ops.segment_sum(jnp.ones((src.shape[0],), jnp.float32), dst,
                                num_segments=N_TERM)
        cnts.append(jnp.broadcast_to(c[:, None], (N_TERM, CNTW)))
    return jnp.stack(sums), jnp.stack(cnts)


def kernel(x_term, x_symbol, x_var, has_arg_src, has_arg_dst, symbol_of_src,
           symbol_of_dst, var_occ_src, var_occ_dst, Wl, bl, Wr, ln_g, ln_b,
           attn_in_w, attn_in_b, attn_out_w, attn_out_b, post_w, post_b):
    sums, cnts = _segment_sums(x_term, x_symbol, x_var, has_arg_src,
                               has_arg_dst, symbol_of_src, symbol_of_dst,
                               var_occ_src, var_occ_dst)
    Wr_sum = Wr[0] + Wr[1] + Wr[2] + Wr[3]
    b_sum = (bl[0] + bl[1] + bl[2] + bl[3]).reshape(1, HID)
    h, q, k, v = _dense_a(sums, cnts, x_term, Wl[:4], Wr_sum, b_sum,
                          ln_g.reshape(1, HID), ln_b.reshape(1, HID),
                          attn_in_w, attn_in_b.reshape(1, 3 * HID))
    return _attn(q, k, v, h, attn_out_w, attn_out_b.reshape(1, HID),
                 post_w, post_b.reshape(1, HID))
